# baseline (speedup 1.0000x reference)
import functools

import jax
import jax.numpy as jnp
from jax import lax
from jax.experimental import pallas as pl
from jax.experimental.pallas import tpu as pltpu
from jax.experimental.pallas import tpu_sc as plsc

_EMB_SIZE = 1000000
_EMB_DIM = 64
_BATCH = 16384
_N_NEG = 5
_NW = 32
_CHUNK = 64
_BLOCKS = 8
_BPW = _BATCH // _NW

_REPACK_ROWS = 4000


_PW = 512
_NPAIR = 977
_T128_ROWS = _NPAIR * _PW


def _repack_body(a_ref, b_ref, o_ref):
    o_ref[:, :_EMB_DIM] = jnp.transpose(a_ref[...])
    o_ref[:, _EMB_DIM:] = jnp.transpose(b_ref[...])


def _tc_repack(table_t):
    return pl.pallas_call(
        _repack_body,
        grid=(_NPAIR,),
        in_specs=[
            pl.BlockSpec((_EMB_DIM, _PW), lambda i: (0, 2 * i)),
            pl.BlockSpec((_EMB_DIM, _PW), lambda i: (0, 2 * i + 1)),
        ],
        out_specs=pl.BlockSpec((_PW, 2 * _EMB_DIM), lambda i: (i, 0)),
        out_shape=jax.ShapeDtypeStruct((_T128_ROWS, 2 * _EMB_DIM),
                                       jnp.float32),
    )(table_t, table_t)


def _log_sigmoid_vec(x):
    e = jnp.exp(-jnp.abs(x))
    t = e / (2.0 + e)
    t2 = t * t
    log1pe = 2.0 * t * (1.0 + t2 * (1.0 / 3.0 + t2 * (0.2 + t2 * (1.0 / 7.0 + t2 * (1.0 / 9.0)))))
    return jnp.minimum(x, 0.0) - log1pe


def _sc_loss_partials(t128, iu2, iv2, ineg2, pu, pv, pn):
    mesh = plsc.VectorSubcoreMesh(core_axis_name="c", subcore_axis_name="s")
    info = plsc.get_sparse_core_info()
    nc = info.num_cores

    @functools.partial(
        pl.kernel,
        mesh=mesh,
        out_type=jax.ShapeDtypeStruct((_NW, 16), jnp.float32),
        scratch_types=[
            pltpu.VMEM((_BLOCKS, _CHUNK), jnp.int32),
            pltpu.VMEM((_BLOCKS, _CHUNK), jnp.int32),
            pltpu.VMEM((_BLOCKS * _N_NEG, _CHUNK), jnp.int32),
            pltpu.VMEM((_BPW,), jnp.int32),
            pltpu.VMEM((_BPW,), jnp.int32),
            pltpu.VMEM((_BPW * _N_NEG,), jnp.int32),
            pltpu.VMEM((_CHUNK, 2 * _EMB_DIM), jnp.float32),
            pltpu.VMEM((_CHUNK, 2 * _EMB_DIM), jnp.float32),
            pltpu.VMEM((_CHUNK, 2 * _EMB_DIM), jnp.float32),
            pltpu.VMEM((_CHUNK, 2 * _EMB_DIM), jnp.float32),
            pltpu.VMEM((_CHUNK * _N_NEG, 2 * _EMB_DIM), jnp.float32),
            pltpu.VMEM((_CHUNK * _N_NEG, 2 * _EMB_DIM), jnp.float32),
            pltpu.VMEM((16,), jnp.float32),
            pltpu.SemaphoreType.DMA,
        ],
        compiler_params=pltpu.CompilerParams(use_tc_tiling_on_sc=False,
                                             needs_layout_passes=False,
                                             disable_bounds_checks=True),
    )
    def k(t_hbm, iu_hbm, iv_hbm, ineg_hbm, pu_hbm, pv_hbm, pn_hbm, out,
          iu_v, iv_v, ineg_v, pu_v, pv_v, pn_v,
          uA, uB, vA, vB, nA, nB, acc_v, sem):
        wid = lax.axis_index("s") * nc + lax.axis_index("c")
        pltpu.sync_copy(iu_hbm.at[wid], iu_v)
        pltpu.sync_copy(iv_hbm.at[wid], iv_v)
        pltpu.sync_copy(ineg_hbm.at[wid], ineg_v)
        pltpu.sync_copy(pu_hbm.at[wid], pu_v)
        pltpu.sync_copy(pv_hbm.at[wid], pv_v)
        pltpu.sync_copy(pn_hbm.at[wid], pn_v)

        def fire(t, ub, vb, nb):
            cps = [
                pltpu.async_copy(t_hbm.at[iu_v.at[t]], ub, sem),
                pltpu.async_copy(t_hbm.at[iv_v.at[t]], vb, sem),
            ]
            for n5 in range(_N_NEG):
                cps.append(pltpu.async_copy(
                    t_hbm.at[ineg_v.at[_N_NEG * t + n5]],
                    nb.at[pl.ds(_CHUNK * n5, _CHUNK)], sem))
            return cps

        iota16 = lax.iota(jnp.int32, 16)

        def compute_block(t, ub, vb, nb, accs):
            def g_body(g, accs):
                acc_p, acc_n = accs
                e = g * 16 + iota16
                e5 = e * _N_NEG
                eoff = t * _CHUNK + g * 16
                bu = pu_v[pl.ds(eoff, 16)] * _EMB_DIM
                bv = pv_v[pl.ds(eoff, 16)] * _EMB_DIM
                bn = [pn_v[pl.ds(n * _BPW + eoff, 16)] * _EMB_DIM
                      for n in range(_N_NEG)]
                zero = jnp.zeros((16,), jnp.float32)

                @plsc.parallel_loop(0, _EMB_DIM, unroll=8,
                                    carry=(zero, zero, zero, zero, zero, zero))
                def dots(d, carry):
                    pos, nd0, nd1, nd2, nd3, nd4 = carry
                    nds = [nd0, nd1, nd2, nd3, nd4]
                    rot = (d + iota16) & (_EMB_DIM - 1)
                    uvec = plsc.load_gather(ub, [e, bu + rot])
                    vvec = plsc.load_gather(vb, [e, bv + rot])
                    pos = pos + uvec * vvec
                    for n in range(_N_NEG):
                        nvec = plsc.load_gather(nb, [e5 + n, bn[n] + rot])
                        nds[n] = nds[n] + nvec * uvec
                    return (pos, nds[0], nds[1], nds[2], nds[3], nds[4])

                acc_p = acc_p + _log_sigmoid_vec(dots[0])
                for n in range(_N_NEG):
                    acc_n = acc_n + _log_sigmoid_vec(-dots[1 + n])
                return (acc_p, acc_n)

            return lax.fori_loop(0, _CHUNK // 16, g_body, accs)

        zero = jnp.zeros((16,), jnp.float32)
        accs = (zero, zero)
        cps = fire(0, uA, vA, nA)
        for t in range(_BLOCKS):
            for cp in cps:
                cp.wait()
            cur = (uA, vA, nA) if t % 2 == 0 else (uB, vB, nB)
            if t + 1 < _BLOCKS:
                nxt = (uB, vB, nB) if t % 2 == 0 else (uA, vA, nA)
                cps = fire(t + 1, *nxt)
            accs = compute_block(t, *cur, accs)

        acc = accs[0] * (1.0 / _BATCH) + accs[1] * (1.0 / (_BATCH * _N_NEG))
        acc_v[...] = acc
        pltpu.sync_copy(acc_v, out.at[wid])

    return k(t128, iu2, iv2, ineg2, pu, pv, pn)


def _pair_split(i):
    return ((i >> 10) << 9) + (i & (_PW - 1)), (i >> 9) & 1


def kernel(pos_u, pos_v, neg_v, u_embeddings):
    t128 = _tc_repack(u_embeddings.T)
    iu2_f, pu_f = _pair_split(pos_u)
    iv2_f, pv_f = _pair_split(pos_v)
    in2_f, pn_f = _pair_split(neg_v)
    iu2 = iu2_f.reshape(_NW, _BLOCKS, _CHUNK)
    iv2 = iv2_f.reshape(_NW, _BLOCKS, _CHUNK)
    ineg2 = in2_f.reshape(_NW, _BLOCKS * _N_NEG, _CHUNK)
    pu = pu_f.reshape(_NW, _BPW)
    pv = pv_f.reshape(_NW, _BPW)
    pn = (pn_f.reshape(_NW, _BPW, _N_NEG)
          .transpose(0, 2, 1).reshape(_NW, _BPW * _N_NEG))
    partials = _sc_loss_partials(t128, iu2, iv2, ineg2, pu, pv, pn)
    return -jnp.sum(partials)


# speedup vs baseline: 2.5682x; 2.5682x over previous
import functools

import jax
import jax.numpy as jnp
from jax import lax
from jax.experimental import pallas as pl
from jax.experimental.pallas import tpu as pltpu
from jax.experimental.pallas import tpu_sc as plsc

_EMB_SIZE = 1000000
_EMB_DIM = 64
_BATCH = 16384
_N_NEG = 5
_NW = 32
_CHUNK = 64
_BLOCKS = 8
_BPW = _BATCH // _NW

_REPACK_ROWS = 4000


_PW = 512
_PAIRS_PER_STEP = 16
_STEP_COLS = 2 * _PW * _PAIRS_PER_STEP
_REPACK_GRID = 62
_T128_ROWS = _REPACK_GRID * _PAIRS_PER_STEP * _PW


def _repack_body(x_ref, o_ref):
    for p in range(_PAIRS_PER_STEP):
        a = x_ref[:, p * 2 * _PW: p * 2 * _PW + _PW]
        b = x_ref[:, p * 2 * _PW + _PW: (p + 1) * 2 * _PW]
        o_ref[p * _PW:(p + 1) * _PW, :_EMB_DIM] = jnp.transpose(a)
        o_ref[p * _PW:(p + 1) * _PW, _EMB_DIM:] = jnp.transpose(b)


def _tc_repack(table_t):
    return pl.pallas_call(
        _repack_body,
        grid=(_REPACK_GRID,),
        in_specs=[pl.BlockSpec((_EMB_DIM, _STEP_COLS), lambda i: (0, i))],
        out_specs=pl.BlockSpec((_PAIRS_PER_STEP * _PW, 2 * _EMB_DIM),
                               lambda i: (i, 0)),
        out_shape=jax.ShapeDtypeStruct((_T128_ROWS, 2 * _EMB_DIM),
                                       jnp.float32),
    )(table_t)


def _log_sigmoid_vec(x):
    e = jnp.exp(-jnp.abs(x))
    t = e / (2.0 + e)
    t2 = t * t
    log1pe = 2.0 * t * (1.0 + t2 * (1.0 / 3.0 + t2 * (0.2 + t2 * (1.0 / 7.0 + t2 * (1.0 / 9.0)))))
    return jnp.minimum(x, 0.0) - log1pe


def _sc_loss_partials(t128, iu2, iv2, ineg2, pu, pv, pn):
    mesh = plsc.VectorSubcoreMesh(core_axis_name="c", subcore_axis_name="s")
    info = plsc.get_sparse_core_info()
    nc = info.num_cores

    @functools.partial(
        pl.kernel,
        mesh=mesh,
        out_type=jax.ShapeDtypeStruct((_NW, 16), jnp.float32),
        scratch_types=[
            pltpu.VMEM((_BLOCKS, _CHUNK), jnp.int32),
            pltpu.VMEM((_BLOCKS, _CHUNK), jnp.int32),
            pltpu.VMEM((_BLOCKS * _N_NEG, _CHUNK), jnp.int32),
            pltpu.VMEM((_BPW,), jnp.int32),
            pltpu.VMEM((_BPW,), jnp.int32),
            pltpu.VMEM((_BPW * _N_NEG,), jnp.int32),
            pltpu.VMEM((_CHUNK, 2 * _EMB_DIM), jnp.float32),
            pltpu.VMEM((_CHUNK, 2 * _EMB_DIM), jnp.float32),
            pltpu.VMEM((_CHUNK, 2 * _EMB_DIM), jnp.float32),
            pltpu.VMEM((_CHUNK, 2 * _EMB_DIM), jnp.float32),
            pltpu.VMEM((_CHUNK * _N_NEG, 2 * _EMB_DIM), jnp.float32),
            pltpu.VMEM((_CHUNK * _N_NEG, 2 * _EMB_DIM), jnp.float32),
            pltpu.VMEM((16,), jnp.float32),
            pltpu.SemaphoreType.DMA,
        ],
        compiler_params=pltpu.CompilerParams(use_tc_tiling_on_sc=False,
                                             needs_layout_passes=False,
                                             disable_bounds_checks=True),
    )
    def k(t_hbm, iu_hbm, iv_hbm, ineg_hbm, pu_hbm, pv_hbm, pn_hbm, out,
          iu_v, iv_v, ineg_v, pu_v, pv_v, pn_v,
          uA, uB, vA, vB, nA, nB, acc_v, sem):
        wid = lax.axis_index("s") * nc + lax.axis_index("c")
        pltpu.sync_copy(iu_hbm.at[wid], iu_v)
        pltpu.sync_copy(iv_hbm.at[wid], iv_v)
        pltpu.sync_copy(ineg_hbm.at[wid], ineg_v)
        pltpu.sync_copy(pu_hbm.at[wid], pu_v)
        pltpu.sync_copy(pv_hbm.at[wid], pv_v)
        pltpu.sync_copy(pn_hbm.at[wid], pn_v)

        def fire(t, ub, vb, nb):
            cps = [
                pltpu.async_copy(t_hbm.at[iu_v.at[t]], ub, sem),
                pltpu.async_copy(t_hbm.at[iv_v.at[t]], vb, sem),
            ]
            for n5 in range(_N_NEG):
                cps.append(pltpu.async_copy(
                    t_hbm.at[ineg_v.at[_N_NEG * t + n5]],
                    nb.at[pl.ds(_CHUNK * n5, _CHUNK)], sem))
            return cps

        iota16 = lax.iota(jnp.int32, 16)

        def compute_block(t, ub, vb, nb, accs):
            def g_body(g, accs):
                acc_p, acc_n = accs
                e = g * 16 + iota16
                e5 = e * _N_NEG
                eoff = t * _CHUNK + g * 16
                bu = pu_v[pl.ds(eoff, 16)] * _EMB_DIM
                bv = pv_v[pl.ds(eoff, 16)] * _EMB_DIM
                bn = [pn_v[pl.ds(n * _BPW + eoff, 16)] * _EMB_DIM
                      for n in range(_N_NEG)]
                zero = jnp.zeros((16,), jnp.float32)

                @plsc.parallel_loop(0, _EMB_DIM, unroll=8,
                                    carry=(zero, zero, zero, zero, zero, zero))
                def dots(d, carry):
                    pos, nd0, nd1, nd2, nd3, nd4 = carry
                    nds = [nd0, nd1, nd2, nd3, nd4]
                    rot = (d + iota16) & (_EMB_DIM - 1)
                    uvec = plsc.load_gather(ub, [e, bu + rot])
                    vvec = plsc.load_gather(vb, [e, bv + rot])
                    pos = pos + uvec * vvec
                    for n in range(_N_NEG):
                        nvec = plsc.load_gather(nb, [e5 + n, bn[n] + rot])
                        nds[n] = nds[n] + nvec * uvec
                    return (pos, nds[0], nds[1], nds[2], nds[3], nds[4])

                acc_p = acc_p + _log_sigmoid_vec(dots[0])
                for n in range(_N_NEG):
                    acc_n = acc_n + _log_sigmoid_vec(-dots[1 + n])
                return (acc_p, acc_n)

            return lax.fori_loop(0, _CHUNK // 16, g_body, accs)

        zero = jnp.zeros((16,), jnp.float32)
        accs = (zero, zero)
        cps = fire(0, uA, vA, nA)
        for t in range(_BLOCKS):
            for cp in cps:
                cp.wait()
            cur = (uA, vA, nA) if t % 2 == 0 else (uB, vB, nB)
            if t + 1 < _BLOCKS:
                nxt = (uB, vB, nB) if t % 2 == 0 else (uA, vA, nA)
                cps = fire(t + 1, *nxt)
            accs = compute_block(t, *cur, accs)

        acc = accs[0] * (1.0 / _BATCH) + accs[1] * (1.0 / (_BATCH * _N_NEG))
        acc_v[...] = acc
        pltpu.sync_copy(acc_v, out.at[wid])

    return k(t128, iu2, iv2, ineg2, pu, pv, pn)


def _pair_split(i):
    return ((i >> 10) << 9) + (i & (_PW - 1)), (i >> 9) & 1


def kernel(pos_u, pos_v, neg_v, u_embeddings):
    t128 = _tc_repack(u_embeddings.T)
    iu2_f, pu_f = _pair_split(pos_u)
    iv2_f, pv_f = _pair_split(pos_v)
    in2_f, pn_f = _pair_split(neg_v)
    iu2 = iu2_f.reshape(_NW, _BLOCKS, _CHUNK)
    iv2 = iv2_f.reshape(_NW, _BLOCKS, _CHUNK)
    ineg2 = in2_f.reshape(_NW, _BLOCKS * _N_NEG, _CHUNK)
    pu = pu_f.reshape(_NW, _BPW)
    pv = pv_f.reshape(_NW, _BPW)
    pn = (pn_f.reshape(_NW, _BPW, _N_NEG)
          .transpose(0, 2, 1).reshape(_NW, _BPW * _N_NEG))
    partials = _sc_loss_partials(t128, iu2, iv2, ineg2, pu, pv, pn)
    return -jnp.sum(partials)


# speedup vs baseline: 2.7008x; 1.0516x over previous
import functools

import jax
import jax.numpy as jnp
from jax import lax
from jax.experimental import pallas as pl
from jax.experimental.pallas import tpu as pltpu
from jax.experimental.pallas import tpu_sc as plsc

_EMB_SIZE = 1000000
_EMB_DIM = 64
_BATCH = 16384
_N_NEG = 5
_NW = 32
_CHUNK = 64
_BLOCKS = 8
_BPW = _BATCH // _NW

_REPACK_ROWS = 4000


_PW = 512
_PAIRS_PER_STEP = 32
_STEP_COLS = 2 * _PW * _PAIRS_PER_STEP
_REPACK_GRID = 31
_T128_ROWS = _REPACK_GRID * _PAIRS_PER_STEP * _PW


def _repack_body(x_ref, o_ref):
    for p in range(_PAIRS_PER_STEP):
        a = x_ref[:, p * 2 * _PW: p * 2 * _PW + _PW]
        b = x_ref[:, p * 2 * _PW + _PW: (p + 1) * 2 * _PW]
        o_ref[p * _PW:(p + 1) * _PW, :_EMB_DIM] = jnp.transpose(a)
        o_ref[p * _PW:(p + 1) * _PW, _EMB_DIM:] = jnp.transpose(b)


def _tc_repack(table_t):
    return pl.pallas_call(
        _repack_body,
        grid=(_REPACK_GRID,),
        in_specs=[pl.BlockSpec((_EMB_DIM, _STEP_COLS), lambda i: (0, i))],
        out_specs=pl.BlockSpec((_PAIRS_PER_STEP * _PW, 2 * _EMB_DIM),
                               lambda i: (i, 0)),
        out_shape=jax.ShapeDtypeStruct((_T128_ROWS, 2 * _EMB_DIM),
                                       jnp.float32),
    )(table_t)


def _log_sigmoid_vec(x):
    e = jnp.exp(-jnp.abs(x))
    t = e / (2.0 + e)
    t2 = t * t
    log1pe = 2.0 * t * (1.0 + t2 * (1.0 / 3.0 + t2 * (0.2 + t2 * (1.0 / 7.0 + t2 * (1.0 / 9.0)))))
    return jnp.minimum(x, 0.0) - log1pe


def _sc_loss_partials(t128, iu2, iv2, ineg2, pu, pv, pn):
    mesh = plsc.VectorSubcoreMesh(core_axis_name="c", subcore_axis_name="s")
    info = plsc.get_sparse_core_info()
    nc = info.num_cores

    @functools.partial(
        pl.kernel,
        mesh=mesh,
        out_type=jax.ShapeDtypeStruct((_NW, 16), jnp.float32),
        scratch_types=[
            pltpu.VMEM((_BLOCKS, _CHUNK), jnp.int32),
            pltpu.VMEM((_BLOCKS, _CHUNK), jnp.int32),
            pltpu.VMEM((_BLOCKS * _N_NEG, _CHUNK), jnp.int32),
            pltpu.VMEM((_BPW,), jnp.int32),
            pltpu.VMEM((_BPW,), jnp.int32),
            pltpu.VMEM((_BPW * _N_NEG,), jnp.int32),
            pltpu.VMEM((_CHUNK, 2 * _EMB_DIM), jnp.float32),
            pltpu.VMEM((_CHUNK, 2 * _EMB_DIM), jnp.float32),
            pltpu.VMEM((_CHUNK, 2 * _EMB_DIM), jnp.float32),
            pltpu.VMEM((_CHUNK, 2 * _EMB_DIM), jnp.float32),
            pltpu.VMEM((_CHUNK * _N_NEG, 2 * _EMB_DIM), jnp.float32),
            pltpu.VMEM((_CHUNK * _N_NEG, 2 * _EMB_DIM), jnp.float32),
            pltpu.VMEM((16,), jnp.float32),
            pltpu.SemaphoreType.DMA,
        ],
        compiler_params=pltpu.CompilerParams(use_tc_tiling_on_sc=False,
                                             needs_layout_passes=False,
                                             disable_bounds_checks=True),
    )
    def k(t_hbm, iu_hbm, iv_hbm, ineg_hbm, pu_hbm, pv_hbm, pn_hbm, out,
          iu_v, iv_v, ineg_v, pu_v, pv_v, pn_v,
          uA, uB, vA, vB, nA, nB, acc_v, sem):
        wid = lax.axis_index("s") * nc + lax.axis_index("c")
        pltpu.sync_copy(iu_hbm.at[wid], iu_v)
        pltpu.sync_copy(iv_hbm.at[wid], iv_v)
        pltpu.sync_copy(ineg_hbm.at[wid], ineg_v)
        pltpu.sync_copy(pu_hbm.at[wid], pu_v)
        pltpu.sync_copy(pv_hbm.at[wid], pv_v)
        pltpu.sync_copy(pn_hbm.at[wid], pn_v)

        def fire(t, ub, vb, nb):
            cps = [
                pltpu.async_copy(t_hbm.at[iu_v.at[t]], ub, sem),
                pltpu.async_copy(t_hbm.at[iv_v.at[t]], vb, sem),
            ]
            for n5 in range(_N_NEG):
                cps.append(pltpu.async_copy(
                    t_hbm.at[ineg_v.at[_N_NEG * t + n5]],
                    nb.at[pl.ds(_CHUNK * n5, _CHUNK)], sem))
            return cps

        iota16 = lax.iota(jnp.int32, 16)

        def compute_block(t, ub, vb, nb, accs):
            def g_body(g, accs):
                acc_p, acc_n = accs
                e = g * 16 + iota16
                e5 = e * _N_NEG
                eoff = t * _CHUNK + g * 16
                bu = pu_v[pl.ds(eoff, 16)] * _EMB_DIM
                bv = pv_v[pl.ds(eoff, 16)] * _EMB_DIM
                bn = [pn_v[pl.ds(n * _BPW + eoff, 16)] * _EMB_DIM
                      for n in range(_N_NEG)]
                zero = jnp.zeros((16,), jnp.float32)

                @plsc.parallel_loop(0, _EMB_DIM, unroll=8,
                                    carry=(zero, zero, zero, zero, zero, zero))
                def dots(d, carry):
                    pos, nd0, nd1, nd2, nd3, nd4 = carry
                    nds = [nd0, nd1, nd2, nd3, nd4]
                    rot = (d + iota16) & (_EMB_DIM - 1)
                    uvec = plsc.load_gather(ub, [e, bu + rot])
                    vvec = plsc.load_gather(vb, [e, bv + rot])
                    pos = pos + uvec * vvec
                    for n in range(_N_NEG):
                        nvec = plsc.load_gather(nb, [e5 + n, bn[n] + rot])
                        nds[n] = nds[n] + nvec * uvec
                    return (pos, nds[0], nds[1], nds[2], nds[3], nds[4])

                acc_p = acc_p + _log_sigmoid_vec(dots[0])
                for n in range(_N_NEG):
                    acc_n = acc_n + _log_sigmoid_vec(-dots[1 + n])
                return (acc_p, acc_n)

            return lax.fori_loop(0, _CHUNK // 16, g_body, accs)

        zero = jnp.zeros((16,), jnp.float32)
        accs = (zero, zero)
        cps = fire(0, uA, vA, nA)
        for t in range(_BLOCKS):
            for cp in cps:
                cp.wait()
            cur = (uA, vA, nA) if t % 2 == 0 else (uB, vB, nB)
            if t + 1 < _BLOCKS:
                nxt = (uB, vB, nB) if t % 2 == 0 else (uA, vA, nA)
                cps = fire(t + 1, *nxt)
            accs = compute_block(t, *cur, accs)

        acc = accs[0] * (1.0 / _BATCH) + accs[1] * (1.0 / (_BATCH * _N_NEG))
        acc_v[...] = acc
        pltpu.sync_copy(acc_v, out.at[wid])

    return k(t128, iu2, iv2, ineg2, pu, pv, pn)


def _pair_split(i):
    return ((i >> 10) << 9) + (i & (_PW - 1)), (i >> 9) & 1


def kernel(pos_u, pos_v, neg_v, u_embeddings):
    t128 = _tc_repack(u_embeddings.T)
    iu2_f, pu_f = _pair_split(pos_u)
    iv2_f, pv_f = _pair_split(pos_v)
    in2_f, pn_f = _pair_split(neg_v)
    iu2 = iu2_f.reshape(_NW, _BLOCKS, _CHUNK)
    iv2 = iv2_f.reshape(_NW, _BLOCKS, _CHUNK)
    ineg2 = in2_f.reshape(_NW, _BLOCKS * _N_NEG, _CHUNK)
    pu = pu_f.reshape(_NW, _BPW)
    pv = pv_f.reshape(_NW, _BPW)
    pn = (pn_f.reshape(_NW, _BPW, _N_NEG)
          .transpose(0, 2, 1).reshape(_NW, _BPW * _N_NEG))
    partials = _sc_loss_partials(t128, iu2, iv2, ineg2, pu, pv, pn)
    return -jnp.sum(partials)


# speedup vs baseline: 2.7028x; 1.0007x over previous
import functools

import jax
import jax.numpy as jnp
from jax import lax
from jax.experimental import pallas as pl
from jax.experimental.pallas import tpu as pltpu
from jax.experimental.pallas import tpu_sc as plsc

_EMB_SIZE = 1000000
_EMB_DIM = 64
_BATCH = 16384
_N_NEG = 5
_NW = 32
_CHUNK = 64
_BLOCKS = 8
_BPW = _BATCH // _NW

_PW = 512
_PAIRS_PER_STEP = 32
_STEP_COLS = 2 * _PW * _PAIRS_PER_STEP
_REPACK_GRID = 31
_T128_ROWS = _REPACK_GRID * _PAIRS_PER_STEP * _PW


def _repack_body(x_ref, o_ref):
    for p in range(_PAIRS_PER_STEP):
        a = x_ref[:, p * 2 * _PW: p * 2 * _PW + _PW]
        b = x_ref[:, p * 2 * _PW + _PW: (p + 1) * 2 * _PW]
        o_ref[p * _PW:(p + 1) * _PW, :_EMB_DIM] = jnp.transpose(a)
        o_ref[p * _PW:(p + 1) * _PW, _EMB_DIM:] = jnp.transpose(b)


def _tc_repack(table_t):
    return pl.pallas_call(
        _repack_body,
        grid=(_REPACK_GRID,),
        in_specs=[pl.BlockSpec((_EMB_DIM, _STEP_COLS), lambda i: (0, i))],
        out_specs=pl.BlockSpec((_PAIRS_PER_STEP * _PW, 2 * _EMB_DIM),
                               lambda i: (i, 0)),
        out_shape=jax.ShapeDtypeStruct((_T128_ROWS, 2 * _EMB_DIM),
                                       jnp.float32),
    )(table_t)


def _log_sigmoid_vec(x):
    e = jnp.exp(-jnp.abs(x))
    t = e / (2.0 + e)
    t2 = t * t
    log1pe = 2.0 * t * (1.0 + t2 * (1.0 / 3.0 + t2 * (0.2 + t2 * (1.0 / 7.0 + t2 * (1.0 / 9.0)))))
    return jnp.minimum(x, 0.0) - log1pe


def _sc_loss_partials(t128, iu2, iv2, ineg2, pu, pv, pn):
    mesh = plsc.VectorSubcoreMesh(core_axis_name="c", subcore_axis_name="s")
    info = plsc.get_sparse_core_info()
    nc = info.num_cores

    @functools.partial(
        pl.kernel,
        mesh=mesh,
        out_type=jax.ShapeDtypeStruct((_NW, 16), jnp.float32),
        scratch_types=[
            pltpu.VMEM((_BLOCKS, _CHUNK), jnp.int32),
            pltpu.VMEM((_BLOCKS, _CHUNK), jnp.int32),
            pltpu.VMEM((_BLOCKS * _N_NEG, _CHUNK), jnp.int32),
            pltpu.VMEM((_BPW,), jnp.int32),
            pltpu.VMEM((_BPW,), jnp.int32),
            pltpu.VMEM((_BPW * _N_NEG,), jnp.int32),
            pltpu.VMEM((_CHUNK, 2 * _EMB_DIM), jnp.float32),
            pltpu.VMEM((_CHUNK, 2 * _EMB_DIM), jnp.float32),
            pltpu.VMEM((_CHUNK, 2 * _EMB_DIM), jnp.float32),
            pltpu.VMEM((_CHUNK, 2 * _EMB_DIM), jnp.float32),
            pltpu.VMEM((_CHUNK * _N_NEG, 2 * _EMB_DIM), jnp.float32),
            pltpu.VMEM((_CHUNK * _N_NEG, 2 * _EMB_DIM), jnp.float32),
            pltpu.VMEM((16,), jnp.float32),
            pltpu.SemaphoreType.DMA,
        ],
        compiler_params=pltpu.CompilerParams(use_tc_tiling_on_sc=False,
                                             needs_layout_passes=False,
                                             disable_bounds_checks=True),
    )
    def k(t_hbm, iu_hbm, iv_hbm, ineg_hbm, pu_hbm, pv_hbm, pn_hbm, out,
          iu_v, iv_v, ineg_v, pu_v, pv_v, pn_v,
          uA, uB, vA, vB, nA, nB, acc_v, sem):
        wid = lax.axis_index("s") * nc + lax.axis_index("c")
        pltpu.sync_copy(iu_hbm.at[wid], iu_v)
        pltpu.sync_copy(iv_hbm.at[wid], iv_v)
        pltpu.sync_copy(ineg_hbm.at[wid], ineg_v)
        pltpu.sync_copy(pu_hbm.at[wid], pu_v)
        pltpu.sync_copy(pv_hbm.at[wid], pv_v)
        pltpu.sync_copy(pn_hbm.at[wid], pn_v)

        def fire(t, ub, vb, nb):
            cps = [
                pltpu.async_copy(t_hbm.at[iu_v.at[t]], ub, sem),
                pltpu.async_copy(t_hbm.at[iv_v.at[t]], vb, sem),
            ]
            for n5 in range(_N_NEG):
                cps.append(pltpu.async_copy(
                    t_hbm.at[ineg_v.at[_N_NEG * t + n5]],
                    nb.at[pl.ds(_CHUNK * n5, _CHUNK)], sem))
            return cps

        iota16 = lax.iota(jnp.int32, 16)

        def compute_block(t, ub, vb, nb, accs):
            def g_body(g, accs):
                acc_p, acc_n = accs
                e = g * 16 + iota16
                e5 = e * _N_NEG
                eoff = t * _CHUNK + g * 16
                bu = pu_v[pl.ds(eoff, 16)] * _EMB_DIM
                bv = pv_v[pl.ds(eoff, 16)] * _EMB_DIM
                bn = [pn_v[pl.ds(n * _BPW + eoff, 16)] * _EMB_DIM
                      for n in range(_N_NEG)]
                zero = jnp.zeros((16,), jnp.float32)

                @plsc.parallel_loop(0, _EMB_DIM, unroll=8,
                                    carry=(zero, zero, zero, zero, zero, zero))
                def dots(d, carry):
                    pos, nd0, nd1, nd2, nd3, nd4 = carry
                    nds = [nd0, nd1, nd2, nd3, nd4]
                    rot = (d + iota16) & (_EMB_DIM - 1)
                    uvec = plsc.load_gather(ub, [e, bu + rot])
                    vvec = plsc.load_gather(vb, [e, bv + rot])
                    pos = pos + uvec * vvec
                    for n in range(_N_NEG):
                        nvec = plsc.load_gather(nb, [e5 + n, bn[n] + rot])
                        nds[n] = nds[n] + nvec * uvec
                    return (pos, nds[0], nds[1], nds[2], nds[3], nds[4])

                acc_p = acc_p + _log_sigmoid_vec(dots[0])
                for n in range(_N_NEG):
                    acc_n = acc_n + _log_sigmoid_vec(-dots[1 + n])
                return (acc_p, acc_n)

            return lax.fori_loop(0, _CHUNK // 16, g_body, accs)

        zero = jnp.zeros((16,), jnp.float32)
        accs = (zero, zero)
        cps = fire(0, uA, vA, nA)
        for t in range(_BLOCKS):
            for cp in cps:
                cp.wait()
            cur = (uA, vA, nA) if t % 2 == 0 else (uB, vB, nB)
            if t + 1 < _BLOCKS:
                nxt = (uB, vB, nB) if t % 2 == 0 else (uA, vA, nA)
                cps = fire(t + 1, *nxt)
            accs = compute_block(t, *cur, accs)

        acc = accs[0] * (1.0 / _BATCH) + accs[1] * (1.0 / (_BATCH * _N_NEG))
        acc_v[...] = acc
        pltpu.sync_copy(acc_v, out.at[wid])

    return k(t128, iu2, iv2, ineg2, pu, pv, pn)


def _pair_split(i):
    return ((i >> 10) << 9) + (i & (_PW - 1)), (i >> 9) & 1


def kernel(pos_u, pos_v, neg_v, u_embeddings):
    t128 = _tc_repack(u_embeddings.T)
    iu2_f, pu_f = _pair_split(pos_u)
    iv2_f, pv_f = _pair_split(pos_v)
    in2_f, pn_f = _pair_split(neg_v)
    iu2 = iu2_f.reshape(_NW, _BLOCKS, _CHUNK)
    iv2 = iv2_f.reshape(_NW, _BLOCKS, _CHUNK)
    ineg2 = in2_f.reshape(_NW, _BLOCKS * _N_NEG, _CHUNK)
    pu = pu_f.reshape(_NW, _BPW)
    pv = pv_f.reshape(_NW, _BPW)
    pn = (pn_f.reshape(_NW, _BPW, _N_NEG)
          .transpose(0, 2, 1).reshape(_NW, _BPW * _N_NEG))
    partials = _sc_loss_partials(t128, iu2, iv2, ineg2, pu, pv, pn)
    return -jnp.sum(partials)
